# baseline (speedup 1.0000x reference)
import functools

import jax
import jax.numpy as jnp
from jax import lax
from jax.experimental import pallas as pl
from jax.experimental.pallas import tpu as pltpu
from jax.experimental.pallas import tpu_sc as plsc

_NUM_CORES = 2
_NUM_SUBCORES = 16
_NUM_WORKERS = _NUM_CORES * _NUM_SUBCORES


def _sc_batch0(row_embed, w, d):
    rows_per = w // _NUM_WORKERS
    half = rows_per // 2
    mesh = plsc.VectorSubcoreMesh(core_axis_name="c", subcore_axis_name="s")

    @functools.partial(
        pl.kernel,
        mesh=mesh,
        out_type=jax.ShapeDtypeStruct((w, d), row_embed.dtype),
        scratch_types=[
            pltpu.VMEM((half, d), row_embed.dtype),
            pltpu.VMEM((half, d), row_embed.dtype),
            pltpu.SemaphoreType.DMA,
            pltpu.SemaphoreType.DMA,
            pltpu.SemaphoreType.DMA,
        ],
    )
    def _gather(emb_hbm, out_hbm, buf0, buf1, sem_r0, sem_r1, sem_w):
        wid = lax.axis_index("s") * _NUM_CORES + lax.axis_index("c")
        base = wid * rows_per
        r0 = pltpu.async_copy(emb_hbm.at[pl.ds(base, half)], buf0, sem_r0)
        r1 = pltpu.async_copy(emb_hbm.at[pl.ds(base + half, half)], buf1, sem_r1)
        r0.wait()
        w0 = pltpu.async_copy(buf0, out_hbm.at[pl.ds(base, half)], sem_w)
        r1.wait()
        w1 = pltpu.async_copy(buf1, out_hbm.at[pl.ds(base + half, half)], sem_w)
        w0.wait()
        w1.wait()

    return _gather(row_embed)


def kernel(x, row_embed):
    b = x.shape[0]
    w = x.shape[-2]
    d = row_embed.shape[-1]
    block = 2048

    sc_batch0 = _sc_batch0(row_embed, w, d)

    def tc_body(emb_ref, out_ref):
        out_ref[...] = emb_ref[...][None]

    base = pl.pallas_call(
        tc_body,
        grid=(w // block, b - 1),
        in_specs=[pl.BlockSpec((block, d), lambda j, i: (j, 0))],
        out_specs=pl.BlockSpec((1, block, d), lambda j, i: (i + 1, j, 0)),
        out_shape=jax.ShapeDtypeStruct((b, w, d), row_embed.dtype),
    )(row_embed)

    sc_b = lax.optimization_barrier(sc_batch0)
    return lax.dynamic_update_slice(base, sc_b[None], (0, 0, 0))


# speedup vs baseline: 1.2923x; 1.2923x over previous
import functools

import jax
import jax.numpy as jnp
from jax import lax
from jax.experimental import pallas as pl
from jax.experimental.pallas import tpu as pltpu
from jax.experimental.pallas import tpu_sc as plsc

_NUM_CORES = 2
_NUM_SUBCORES = 16
_NUM_WORKERS = _NUM_CORES * _NUM_SUBCORES


def kernel(x, row_embed):
    b = x.shape[0]
    w = x.shape[-2]
    d = row_embed.shape[-1]
    rows_per = w // _NUM_WORKERS

    mesh = plsc.VectorSubcoreMesh(core_axis_name="c", subcore_axis_name="s")

    half = rows_per // 2

    @functools.partial(
        pl.kernel,
        mesh=mesh,
        out_type=jax.ShapeDtypeStruct((b * w, d), row_embed.dtype),
        scratch_types=[
            pltpu.VMEM((half, d), row_embed.dtype),
            pltpu.VMEM((half, d), row_embed.dtype),
            pltpu.SemaphoreType.DMA,
            pltpu.SemaphoreType.DMA,
            pltpu.SemaphoreType.DMA,
        ],
    )
    def _bcast(emb_hbm, out_hbm, buf0, buf1, sem_r0, sem_r1, sem_w):
        wid = lax.axis_index("s") * _NUM_CORES + lax.axis_index("c")
        base = wid * rows_per
        r0 = pltpu.async_copy(emb_hbm.at[pl.ds(base, half)], buf0, sem_r0)
        r1 = pltpu.async_copy(emb_hbm.at[pl.ds(base + half, half)], buf1, sem_r1)
        r0.wait()
        writes = [
            pltpu.async_copy(buf0, out_hbm.at[pl.ds(bb * w + base, half)], sem_w)
            for bb in range(b)
        ]
        r1.wait()
        writes += [
            pltpu.async_copy(buf1, out_hbm.at[pl.ds(bb * w + base + half, half)], sem_w)
            for bb in range(b)
        ]
        for c in writes:
            c.wait()

    return _bcast(row_embed).reshape(b, w, d)
